# baseline (speedup 1.0000x reference)
import functools

import jax
import jax.numpy as jnp
from jax import lax
from jax.experimental import pallas as pl
from jax.experimental.pallas import tpu as pltpu
from jax.experimental.pallas import tpu_sc as plsc

N_SPECIES = 1000000
EMBED_DIM = 32
BATCH = 16384
OUT_DIM = EMBED_DIM + 3

NC = 2
NS = 16
NW = NC * NS
BPW = BATCH // NW
GRP = 8
N_GRP = BPW // GRP


def _body(idx_hbm, g_hbm, s_hbm, c_hbm, tabT_hbm, out_hbm,
          idx_v, blocks_v, g_v, s_v, c_v, out_v, sem):
    c = lax.axis_index("c")
    s = lax.axis_index("s")
    wid = s * NC + c
    base = wid * BPW

    pltpu.sync_copy(idx_hbm.at[pl.ds(base, BPW)], idx_v.at[pl.ds(0, BPW)])
    pltpu.sync_copy(g_hbm.at[pl.ds(base, BPW)], g_v)
    pltpu.sync_copy(s_hbm.at[pl.ds(base, BPW)], s_v)
    pltpu.sync_copy(c_hbm.at[pl.ds(base, BPW)], c_v)

    lanes = lax.iota(jnp.int32, 16)

    def fire(gv, slot0):
        for j in range(GRP):
            r = gv[j]
            blk = lax.shift_right_logical(r, 7)
            off = pl.multiple_of(blk * 128, 128)
            pltpu.async_copy(tabT_hbm.at[:, pl.ds(off, 128)],
                             blocks_v.at[slot0 + j], sem)

    fire(idx_v[pl.ds(0, 16)], 0)
    fire(idx_v[pl.ds(GRP, 16)], GRP)

    def group(g, _):
        gn = jnp.minimum(g + 2, N_GRP - 1)
        fire(idx_v[pl.ds(gn * GRP, 16)], GRP * lax.rem(g + 2, 3))
        for _j in range(GRP):
            pltpu.make_async_copy(tabT_hbm.at[:, pl.ds(0, 128)],
                                  blocks_v.at[0], sem).wait()
        v16 = idx_v[pl.ds(g * GRP, 16)]
        slot0 = GRP * lax.rem(g, 3)
        for j in range(GRP):
            r = v16[j]
            lm = lax.bitwise_and(r, jnp.int32(127))
            jv = jnp.full((16,), 0, jnp.int32) + (slot0 + j)
            lmv = jnp.full((16,), 0, jnp.int32) + lm
            lo = plsc.load_gather(blocks_v, [jv, lanes, lmv])
            hi = plsc.load_gather(blocks_v, [jv, lanes + 16, lmv])
            dst = (g * GRP + j) * OUT_DIM + lanes
            plsc.store_scatter(out_v, [dst], lo)
            plsc.store_scatter(out_v, [dst + 16], hi)
        return 0

    lax.fori_loop(0, N_GRP, group, 0, unroll=False)
    for _j in range(2 * GRP):
        pltpu.make_async_copy(tabT_hbm.at[:, pl.ds(0, 128)],
                              blocks_v.at[0], sem).wait()

    for gblk in range(BPW // 16):
        dst = (16 * gblk + lanes) * OUT_DIM + EMBED_DIM
        plsc.store_scatter(out_v, [dst], g_v[pl.ds(16 * gblk, 16)])
        plsc.store_scatter(out_v, [dst + 1], s_v[pl.ds(16 * gblk, 16)])
        plsc.store_scatter(out_v, [dst + 2], c_v[pl.ds(16 * gblk, 16)])

    pltpu.sync_copy(out_v, out_hbm.at[pl.ds(base * OUT_DIM, BPW * OUT_DIM)])


@functools.partial(jax.jit, static_argnums=())
def kernel(species_idx, group_idx, sin_date, cos_date, species_embedding):
    mesh = plsc.VectorSubcoreMesh(core_axis_name="c", subcore_axis_name="s")
    run = pl.kernel(
        _body,
        mesh=mesh,
        compiler_params=pltpu.CompilerParams(needs_layout_passes=False),
        out_type=jax.ShapeDtypeStruct((BATCH * OUT_DIM,), jnp.float32),
        scratch_types=[
            pltpu.VMEM((BPW + 16,), jnp.int32),
            pltpu.VMEM((3 * GRP, EMBED_DIM, 128), jnp.float32),
            pltpu.VMEM((BPW,), jnp.float32),
            pltpu.VMEM((BPW,), jnp.float32),
            pltpu.VMEM((BPW,), jnp.float32),
            pltpu.VMEM((BPW * OUT_DIM,), jnp.float32),
            pltpu.SemaphoreType.DMA,
        ],
    )
    flat = run(species_idx.astype(jnp.int32), group_idx, sin_date, cos_date,
               species_embedding.T)
    return flat.reshape(BATCH, OUT_DIM)


# speedup vs baseline: 1.0248x; 1.0248x over previous
import functools

import jax
import jax.numpy as jnp
from jax import lax
from jax.experimental import pallas as pl
from jax.experimental.pallas import tpu as pltpu
from jax.experimental.pallas import tpu_sc as plsc

N_SPECIES = 1000000
EMBED_DIM = 32
BATCH = 16384
OUT_DIM = EMBED_DIM + 3

NC = 2
NS = 16
NW = NC * NS
BPW = BATCH // NW
GRP = 8
N_GRP = BPW // GRP


def _body(idx_hbm, g_hbm, s_hbm, c_hbm, tabT_hbm, out_hbm,
          idx_v, blocks_v, g_v, s_v, c_v, out_v, sem):
    c = lax.axis_index("c")
    s = lax.axis_index("s")
    wid = s * NC + c
    base = wid * BPW

    pltpu.sync_copy(idx_hbm.at[pl.ds(base, BPW)], idx_v.at[pl.ds(0, BPW)])
    pltpu.sync_copy(g_hbm.at[pl.ds(base, BPW)], g_v)
    pltpu.sync_copy(s_hbm.at[pl.ds(base, BPW)], s_v)
    pltpu.sync_copy(c_hbm.at[pl.ds(base, BPW)], c_v)

    lanes = lax.iota(jnp.int32, 16)

    def fire(gv, slot0):
        for j in range(GRP):
            r = gv[j]
            blk = lax.shift_right_logical(r, 7)
            off = pl.multiple_of(blk * 128, 128)
            pltpu.async_copy(tabT_hbm.at[:, pl.ds(off, 128)],
                             blocks_v.at[slot0 + j], sem)

    fire(idx_v[pl.ds(0, 16)], 0)

    def group(g, _):
        gn = jnp.minimum(g + 1, N_GRP - 1)
        fire(idx_v[pl.ds(gn * GRP, 16)], GRP * lax.rem(g + 1, 2))
        for _j in range(GRP):
            pltpu.make_async_copy(tabT_hbm.at[:, pl.ds(0, 128)],
                                  blocks_v.at[0], sem).wait()
        v16 = idx_v[pl.ds(g * GRP, 16)]
        slot0 = GRP * lax.rem(g, 2)
        for j in range(GRP):
            r = v16[j]
            lm = lax.bitwise_and(r, jnp.int32(127))
            jv = jnp.full((16,), 0, jnp.int32) + (slot0 + j)
            lmv = jnp.full((16,), 0, jnp.int32) + lm
            lo = plsc.load_gather(blocks_v, [jv, lanes, lmv])
            hi = plsc.load_gather(blocks_v, [jv, lanes + 16, lmv])
            dst = (g * GRP + j) * OUT_DIM + lanes
            plsc.store_scatter(out_v, [dst], lo)
            plsc.store_scatter(out_v, [dst + 16], hi)
        return 0

    lax.fori_loop(0, N_GRP, group, 0, unroll=False)
    for _j in range(GRP):
        pltpu.make_async_copy(tabT_hbm.at[:, pl.ds(0, 128)],
                              blocks_v.at[0], sem).wait()

    for gblk in range(BPW // 16):
        dst = (16 * gblk + lanes) * OUT_DIM + EMBED_DIM
        plsc.store_scatter(out_v, [dst], g_v[pl.ds(16 * gblk, 16)])
        plsc.store_scatter(out_v, [dst + 1], s_v[pl.ds(16 * gblk, 16)])
        plsc.store_scatter(out_v, [dst + 2], c_v[pl.ds(16 * gblk, 16)])

    pltpu.sync_copy(out_v, out_hbm.at[pl.ds(base * OUT_DIM, BPW * OUT_DIM)])


@functools.partial(jax.jit, static_argnums=())
def kernel(species_idx, group_idx, sin_date, cos_date, species_embedding):
    mesh = plsc.VectorSubcoreMesh(core_axis_name="c", subcore_axis_name="s")
    run = pl.kernel(
        _body,
        mesh=mesh,
        compiler_params=pltpu.CompilerParams(needs_layout_passes=False),
        out_type=jax.ShapeDtypeStruct((BATCH * OUT_DIM,), jnp.float32),
        scratch_types=[
            pltpu.VMEM((BPW + 16,), jnp.int32),
            pltpu.VMEM((2 * GRP, EMBED_DIM, 128), jnp.float32),
            pltpu.VMEM((BPW,), jnp.float32),
            pltpu.VMEM((BPW,), jnp.float32),
            pltpu.VMEM((BPW,), jnp.float32),
            pltpu.VMEM((BPW * OUT_DIM,), jnp.float32),
            pltpu.SemaphoreType.DMA,
        ],
    )
    flat = run(species_idx.astype(jnp.int32), group_idx, sin_date, cos_date,
               species_embedding.T)
    return flat.reshape(BATCH, OUT_DIM)
